# baseline (speedup 1.0000x reference)
import functools

import jax
import jax.numpy as jnp
from jax.experimental import pallas as pl
from jax.experimental.pallas import tpu as pltpu

_T = 25000
_K = 2


def _pipeline_kernel(x_hbm, w_ref, o_hbm, in_buf, out_buf, in_sems, out_sems):
    n = x_hbm.shape[0]
    n_steps = n // _T

    def in_copy(i):
        return pltpu.make_async_copy(
            x_hbm.at[pl.ds(i * _T, _T)], in_buf.at[i % _K], in_sems.at[i % _K]
        )

    def out_copy(i):
        return pltpu.make_async_copy(
            out_buf.at[i % _K], o_hbm.at[pl.ds(i * _T, _T)], out_sems.at[i % _K]
        )

    for i in range(min(_K, n_steps)):
        in_copy(i).start()
    for i in range(n_steps):
        in_copy(i).wait()
        if i >= _K:
            out_copy(i - _K).wait()
        out_buf[i % _K] = jax.lax.dot_general(
            in_buf[i % _K], w_ref[...],
            dimension_numbers=(((1,), (1,)), ((), ())),
            preferred_element_type=jnp.float32,
        )
        out_copy(i).start()
        if i + _K < n_steps:
            in_copy(i + _K).start()
    for i in range(max(0, n_steps - _K), n_steps):
        out_copy(i).wait()


@functools.partial(jax.jit, static_argnames=())
def kernel(x, W):
    n, d_in = x.shape
    d_out = W.shape[0]
    return pl.pallas_call(
        _pipeline_kernel,
        in_specs=[
            pl.BlockSpec(memory_space=pl.ANY),
            pl.BlockSpec(memory_space=pltpu.MemorySpace.VMEM),
        ],
        out_specs=pl.BlockSpec(memory_space=pl.ANY),
        out_shape=jax.ShapeDtypeStruct((n, d_out), jnp.float32),
        scratch_shapes=[
            pltpu.VMEM((_K, _T, d_in), jnp.float32),
            pltpu.VMEM((_K, _T, d_out), jnp.float32),
            pltpu.SemaphoreType.DMA((_K,)),
            pltpu.SemaphoreType.DMA((_K,)),
        ],
    )(x, W)


# speedup vs baseline: 1.2446x; 1.2446x over previous
import functools

import jax
import jax.numpy as jnp
from jax.experimental import pallas as pl
from jax.experimental.pallas import tpu as pltpu

_T = 10000
_K = 5


def _pipeline_kernel(x_hbm, w_ref, o_hbm, in_buf, out_buf, in_sems, out_sems):
    n = x_hbm.shape[0]
    n_steps = n // _T

    def in_copy(i):
        return pltpu.make_async_copy(
            x_hbm.at[pl.ds(i * _T, _T)], in_buf.at[i % _K], in_sems.at[i % _K]
        )

    def out_copy(i):
        return pltpu.make_async_copy(
            out_buf.at[i % _K], o_hbm.at[pl.ds(i * _T, _T)], out_sems.at[i % _K]
        )

    for i in range(min(_K, n_steps)):
        in_copy(i).start()
    for i in range(n_steps):
        in_copy(i).wait()
        if i >= _K:
            out_copy(i - _K).wait()
        out_buf[i % _K] = jax.lax.dot_general(
            in_buf[i % _K], w_ref[...],
            dimension_numbers=(((1,), (1,)), ((), ())),
            preferred_element_type=jnp.float32,
        )
        out_copy(i).start()
        if i + _K < n_steps:
            in_copy(i + _K).start()
    for i in range(max(0, n_steps - _K), n_steps):
        out_copy(i).wait()


@functools.partial(jax.jit, static_argnames=())
def kernel(x, W):
    n, d_in = x.shape
    d_out = W.shape[0]
    return pl.pallas_call(
        _pipeline_kernel,
        in_specs=[
            pl.BlockSpec(memory_space=pl.ANY),
            pl.BlockSpec(memory_space=pltpu.MemorySpace.VMEM),
        ],
        out_specs=pl.BlockSpec(memory_space=pl.ANY),
        out_shape=jax.ShapeDtypeStruct((n, d_out), jnp.float32),
        scratch_shapes=[
            pltpu.VMEM((_K, _T, d_in), jnp.float32),
            pltpu.VMEM((_K, _T, d_out), jnp.float32),
            pltpu.SemaphoreType.DMA((_K,)),
            pltpu.SemaphoreType.DMA((_K,)),
        ],
    )(x, W)


# speedup vs baseline: 1.3124x; 1.0545x over previous
import functools

import jax
import jax.numpy as jnp
from jax.experimental import pallas as pl
from jax.experimental.pallas import tpu as pltpu

_BLK = 29952


def _linear_kernel(x_ref, w_ref, o_ref):
    o_ref[...] = jax.lax.dot_general(
        x_ref[...], w_ref[...],
        dimension_numbers=(((1,), (1,)), ((), ())),
        preferred_element_type=jnp.float32,
    )


@functools.partial(jax.jit, static_argnames=())
def kernel(x, W):
    n, d_in = x.shape
    d_out = W.shape[0]
    blk = _BLK
    grid = (pl.cdiv(n, blk),)
    return pl.pallas_call(
        _linear_kernel,
        grid=grid,
        in_specs=[
            pl.BlockSpec((blk, d_in), lambda i: (i, 0)),
            pl.BlockSpec((d_out, d_in), lambda i: (0, 0)),
        ],
        out_specs=pl.BlockSpec((blk, d_out), lambda i: (i, 0)),
        out_shape=jax.ShapeDtypeStruct((n, d_out), jnp.float32),
        compiler_params=pltpu.CompilerParams(
            dimension_semantics=("parallel",),
        ),
    )(x, W)


# speedup vs baseline: 1.3281x; 1.0120x over previous
import functools

import jax
import jax.numpy as jnp
from jax.experimental import pallas as pl
from jax.experimental.pallas import tpu as pltpu

_BLK = 29952


def _linear_kernel(x_ref, w_ref, o_ref):
    o_ref[...] = jax.lax.dot_general(
        x_ref[...], w_ref[...],
        dimension_numbers=(((1,), (1,)), ((), ())),
        preferred_element_type=jnp.float32,
    )


@functools.partial(jax.jit, static_argnames=())
def kernel(x, W):
    n, d_in = x.shape
    d_out = W.shape[0]
    blk = _BLK
    grid = (pl.cdiv(n, blk),)
    return pl.pallas_call(
        _linear_kernel,
        grid=grid,
        in_specs=[
            pl.BlockSpec((blk, d_in), lambda i: (i, 0)),
            pl.BlockSpec((d_out, d_in), lambda i: (0, 0)),
        ],
        out_specs=pl.BlockSpec((blk, d_out), lambda i: (i, 0)),
        out_shape=jax.ShapeDtypeStruct((n, d_out), jnp.float32),
        compiler_params=pltpu.CompilerParams(
            dimension_semantics=("arbitrary",),
        ),
    )(x, W)
